# Initial kernel scaffold; baseline (speedup 1.0000x reference)
import jax
import jax.numpy as jnp
from jax.experimental import pallas as pl


def kernel(logits, codes, codebook):
    raise NotImplementedError("write your pallas kernel here")


# speedup vs baseline: 2.3315x; 2.3315x over previous
import functools

import jax
import jax.numpy as jnp
from jax import lax
from jax.experimental import pallas as pl

_B, _K, _D = 4096, 8192, 32
_BB = 256


def _tc_body(logits_ref, codes_ref, codebook_ref,
             hard_ref, soft_ref, lp_ref, ent_ref):
    x = logits_ref[...]
    m = jnp.max(x, axis=1, keepdims=True)
    e = jnp.exp(x - m)
    s = jnp.sum(e, axis=1, keepdims=True)
    t = jnp.sum(e * x, axis=1, keepdims=True)
    logs = jnp.log(s)

    codes = codes_ref[...]
    iota = lax.broadcasted_iota(jnp.int32, (_BB, _K), 1)
    oh = iota == codes
    l_code = jnp.sum(jnp.where(oh, x, 0.0), axis=1, keepdims=True)

    cb = codebook_ref[...]
    dn = (((1,), (0,)), ((), ()))
    v = lax.dot_general(e, cb, dn, preferred_element_type=jnp.float32)
    h = lax.dot_general(oh.astype(jnp.float32), cb, dn,
                        preferred_element_type=jnp.float32)

    hard_ref[...] = h
    soft_ref[...] = v / s
    lp_ref[...] = l_code - m - logs
    ent_ref[...] = m + logs - t / s


@functools.partial(jax.jit, static_argnames=("interpret",))
def kernel(logits, codes, codebook, interpret=False):
    grid = (_B // _BB,)
    hard, soft, lp, ent = pl.pallas_call(
        _tc_body,
        grid=grid,
        in_specs=[
            pl.BlockSpec((_BB, _K), lambda i: (i, 0)),
            pl.BlockSpec((_BB, 1), lambda i: (i, 0)),
            pl.BlockSpec((_K, _D), lambda i: (0, 0)),
        ],
        out_specs=[
            pl.BlockSpec((_BB, _D), lambda i: (i, 0)),
            pl.BlockSpec((_BB, _D), lambda i: (i, 0)),
            pl.BlockSpec((_BB, 1), lambda i: (i, 0)),
            pl.BlockSpec((_BB, 1), lambda i: (i, 0)),
        ],
        out_shape=[
            jax.ShapeDtypeStruct((_B, _D), jnp.float32),
            jax.ShapeDtypeStruct((_B, _D), jnp.float32),
            jax.ShapeDtypeStruct((_B, 1), jnp.float32),
            jax.ShapeDtypeStruct((_B, 1), jnp.float32),
        ],
        interpret=interpret,
    )(logits, codes.reshape(_B, 1), codebook)
    return jnp.concatenate([hard, soft, lp, ent], axis=-1)
